# Initial kernel scaffold; baseline (speedup 1.0000x reference)
import jax
import jax.numpy as jnp
from jax.experimental import pallas as pl


def kernel(x, edge_index, edge_attr, ne_W0, ne_b0, ne_W1, ne_b1, ee_W0, ee_b0, ee_W1, ee_b1, g_rW0, g_rb0, g_rW1, g_rb1, g_oW0, g_ob0, g_oW1, g_ob1, w_W0, w_b0, w_W1, w_b1):
    raise NotImplementedError("write your pallas kernel here")


# speedup vs baseline: 6.5650x; 6.5650x over previous
import functools

import jax
import jax.numpy as jnp
from jax import lax
from jax.experimental import pallas as pl
from jax.experimental.pallas import tpu as pltpu
from jax.experimental.pallas import tpu_sc as plsc

_DOT = functools.partial(jnp.dot, precision=jax.lax.Precision.HIGHEST)

_NC = 2
_NS = 16
_NW = _NC * _NS
_TE = 1024
_IDXW = 128
_F = 16
_BQ = 4096


def _sc_edge_pass(dst_rows, src_rows, q16, pd16, ps16, *, n_pad, e_pad, iters):
    nps = n_pad // _NS
    epw = e_pad // _NW
    rpw = epw // _IDXW
    k_chunks = _TE // _IDXW
    mesh = plsc.VectorSubcoreMesh(core_axis_name="c", subcore_axis_name="s",
                                  num_cores=_NC, num_subcores=_NS)

    @functools.partial(
        pl.kernel,
        out_type=[
            jax.ShapeDtypeStruct((e_pad, _F), jnp.float32),
            jax.ShapeDtypeStruct((_NC * n_pad, _F), jnp.float32),
        ],
        mesh=mesh,
        compiler_params=pltpu.CompilerParams(use_tc_tiling_on_sc=False),
        scratch_types=[
            pltpu.VMEM_SHARED((n_pad, _F), jnp.float32),
            pltpu.VMEM((k_chunks, _IDXW), jnp.int32),
            pltpu.VMEM((k_chunks, _IDXW), jnp.int32),
            pltpu.VMEM((_TE, _F), jnp.float32),
            pltpu.VMEM((_TE, _F), jnp.float32),
            pltpu.VMEM((_TE, _F), jnp.float32),
            pltpu.SemaphoreType.DMA,
        ],
    )
    def body(dst_r, src_r, q_r, pd_r, ps_r, h_r, s_r, s_acc, idxd, idxs,
             qv, gd, gs, sem):
        cid = lax.axis_index("c")
        sid = lax.axis_index("s")
        w = sid * _NC + cid

        def zbody(i, _):
            qv[i] = jnp.zeros((_F,), jnp.float32)
            return 0
        lax.fori_loop(0, _TE, zbody, 0, unroll=8)
        sbase = sid * nps
        full, rem = nps // _TE, nps % _TE
        for k in range(full):
            pltpu.sync_copy(qv, s_acc.at[pl.ds(sbase + k * _TE, _TE)])
        if rem:
            pltpu.sync_copy(qv.at[pl.ds(0, rem)],
                            s_acc.at[pl.ds(sbase + full * _TE, rem)])
        plsc.subcore_barrier()

        def step(t, _):
            off_e = w * epw + t * _TE
            off_r = w * rpw + t * k_chunks
            pltpu.sync_copy(dst_r.at[pl.ds(off_r, k_chunks)], idxd)
            pltpu.sync_copy(src_r.at[pl.ds(off_r, k_chunks)], idxs)
            pltpu.sync_copy(q_r.at[pl.ds(off_e, _TE)], qv)
            cps = []
            for j in range(k_chunks):
                cps.append(pltpu.async_copy(
                    pd_r.at[idxd.at[j]], gd.at[pl.ds(j * _IDXW, _IDXW)], sem))
                cps.append(pltpu.async_copy(
                    ps_r.at[idxs.at[j]], gs.at[pl.ds(j * _IDXW, _IDXW)], sem))
            for c in cps:
                c.wait()

            def relu_row(i, _):
                gd[i] = jnp.maximum(gd[i] + gs[i] + qv[i], 0.0)
                return 0
            lax.fori_loop(0, _TE, relu_row, 0, unroll=8)

            pltpu.sync_copy(gd, h_r.at[pl.ds(off_e, _TE)])
            for j in range(k_chunks):
                pltpu.sync_copy(gd.at[pl.ds(j * _IDXW, _IDXW)],
                                s_acc.at[idxd.at[j]], add=True)
            return 0
        lax.fori_loop(0, iters, step, 0)

        plsc.subcore_barrier()
        pltpu.sync_copy(s_acc.at[pl.ds(sbase, nps)],
                        s_r.at[pl.ds(cid * n_pad + sbase, nps)])

    return body(dst_rows, src_rows, q16, pd16, ps16)


def _wspec(shape):
    nd = len(shape)
    return pl.BlockSpec(shape, lambda *a: (0,) * nd)


def _grp(w, gin=16, gout=16, reps=8):
    wp = jnp.zeros((gin, gout), jnp.float32)
    wp = wp.at[: w.shape[0], : w.shape[1]].set(w)
    return jnp.kron(jnp.eye(reps, dtype=jnp.float32), wp)


def _qvec(cb, reps=8):
    g = jnp.concatenate([cb, jnp.ones((1,), jnp.float32),
                         jnp.zeros((7,), jnp.float32)])
    return jnp.tile(g, reps).reshape(1, -1)


def _pad_tile(v, width, reps):
    g = jnp.zeros((width,), jnp.float32).at[: v.shape[0]].set(v)
    return jnp.tile(g, reps).reshape(1, -1)


def _enc_edge(ea_v, m0, b0, m1, b1, *, ea_rows):
    blk = 1600
    def body(x_ref, m0_ref, b0_ref, m1_ref, b1_ref, o_ref):
        u = jnp.maximum(_DOT(x_ref[...], m0_ref[...]) + b0_ref[...], 0.0)
        q = _DOT(u, m1_ref[...]) + b1_ref[...]
        for a in range(4):
            o_ref[:, a, :] = q[:, 128 * a:128 * (a + 1)]
    return pl.pallas_call(
        body,
        grid=(ea_rows // blk,),
        in_specs=[pl.BlockSpec((blk, 128), lambda i: (i, 0)),
                  _wspec((128, 256)), _wspec((1, 256)),
                  _wspec((256, 512)), _wspec((1, 512))],
        out_specs=pl.BlockSpec((blk, 4, 128), lambda i: (i, 0, 0)),
        out_shape=jax.ShapeDtypeStruct((ea_rows, 4, 128), jnp.float32),
    )(ea_v, m0, b0, m1, b1)


def _q_stream(h_v, m, b, *, eq_rows):
    def body(x_ref, m_ref, b_ref, o_ref):
        o_ref[...] = _DOT(x_ref[...], m_ref[...]) + b_ref[...]
    return pl.pallas_call(
        body,
        grid=(eq_rows // _BQ,),
        in_specs=[pl.BlockSpec((_BQ, 128), lambda i: (i, 0)),
                  _wspec((128, 128)), _wspec((1, 128))],
        out_specs=pl.BlockSpec((_BQ, 128), lambda i: (i, 0)),
        out_shape=jax.ShapeDtypeStruct((eq_rows, 128), jnp.float32),
    )(h_v, m, b)


def _enc_node(x_v, m0, b0, m1, b1, md, ms, *, nx_rows):
    def body(x_ref, m0_ref, b0_ref, m1_ref, b1_ref, md_ref, ms_ref,
             n_out, pd_out, ps_out):
        u = jnp.maximum(_DOT(x_ref[...], m0_ref[...]) + b0_ref[...], 0.0)
        nn = _DOT(u, m1_ref[...]) + b1_ref[...]
        pd = _DOT(nn, md_ref[...])
        ps = _DOT(nn, ms_ref[...])
        for a in range(4):
            n_out[:, a, :] = nn[:, 128 * a:128 * (a + 1)]
            pd_out[:, a, :] = pd[:, 128 * a:128 * (a + 1)]
            ps_out[:, a, :] = ps[:, 128 * a:128 * (a + 1)]
    spec = pl.BlockSpec((nx_rows, 4, 128), lambda: (0, 0, 0))
    return pl.pallas_call(
        body,
        in_specs=[pl.BlockSpec((nx_rows, 128), lambda: (0, 0)),
                  _wspec((128, 256)), _wspec((1, 256)),
                  _wspec((256, 512)), _wspec((1, 512)),
                  _wspec((512, 512)), _wspec((512, 512))],
        out_specs=[spec, spec, spec],
        out_shape=[jax.ShapeDtypeStruct((nx_rows, 4, 128), jnp.float32)] * 3,
    )(x_v, m0, b0, m1, b1, md, ms)


def _node_update(n_v, s0_v, s1_v, mso, mn, m1, bh, bn, md, ms, *, nq_rows):
    blk = nq_rows // 2
    def body(n_ref, s0_ref, s1_ref, mso_ref, mn_ref, m1_ref, bh_ref, bn_ref,
             md_ref, ms_ref, n_out, pd_out, ps_out):
        ss = s0_ref[...] + s1_ref[...]
        hid = jnp.maximum(
            _DOT(n_ref[...], mn_ref[...]) + _DOT(ss, mso_ref[...])
            + bh_ref[...], 0.0)
        nn = _DOT(hid, m1_ref[...]) + bn_ref[...]
        n_out[...] = nn
        pd_out[...] = _DOT(nn, md_ref[...])
        ps_out[...] = _DOT(nn, ms_ref[...])
    spec = pl.BlockSpec((blk, 128), lambda i: (i, 0))
    return pl.pallas_call(
        body,
        grid=(nq_rows // blk,),
        in_specs=[spec, spec, spec,
                  _wspec((128, 128)), _wspec((128, 128)), _wspec((128, 128)),
                  _wspec((1, 128)), _wspec((1, 128)),
                  _wspec((128, 128)), _wspec((128, 128))],
        out_specs=[spec, spec, spec],
        out_shape=[jax.ShapeDtypeStruct((nq_rows, 128), jnp.float32)] * 3,
    )(n_v, s0_v, s1_v, mso, mn, m1, bh, bn, md, ms)


def _head(hw_v, mw, bw, *, eq_rows):
    def body(x_ref, m_ref, b_ref, o_ref):
        v = _DOT(x_ref[...], m_ref[...]) + b_ref[...]
        o_ref[...] = 1.0 / (1.0 + jnp.exp(-v))
    return pl.pallas_call(
        body,
        grid=(eq_rows // _BQ,),
        in_specs=[pl.BlockSpec((_BQ, 128), lambda i: (i, 0)),
                  _wspec((128, 8)), _wspec((1, 8))],
        out_specs=pl.BlockSpec((_BQ, 8), lambda i: (i, 0)),
        out_shape=jax.ShapeDtypeStruct((eq_rows, 8), jnp.float32),
    )(hw_v, mw, bw)


def kernel(x, edge_index, edge_attr, ne_W0, ne_b0, ne_W1, ne_b1,
           ee_W0, ee_b0, ee_W1, ee_b1, g_rW0, g_rb0, g_rW1, g_rb1,
           g_oW0, g_ob0, g_oW1, g_ob1, w_W0, w_b0, w_W1, w_b1):
    n = x.shape[0]
    e = edge_attr.shape[0]
    tile = _NW * _TE
    iters = -(-e // tile)
    e_pad = iters * tile
    n_pad = -(-(n + 32) // 128) * 128
    pad = e_pad - e
    eq = e_pad // 8
    ea_rows = e_pad // 32
    nx = n_pad // 32
    nq = n_pad // 8

    dump = n + (jnp.arange(pad, dtype=jnp.int32) % 32)
    src_rows = jnp.concatenate([edge_index[0], dump]).reshape(e_pad // _IDXW,
                                                              _IDXW)
    dst_rows = jnp.concatenate([edge_index[1], dump]).reshape(e_pad // _IDXW,
                                                              _IDXW)
    ea_v = jnp.concatenate(
        [edge_attr, jnp.zeros((pad, edge_attr.shape[1]), jnp.float32)]
    ).reshape(ea_rows, 128)
    x4 = jnp.concatenate(
        [x, jnp.zeros((n_pad - n, x.shape[1]), jnp.float32)])
    x4 = jnp.concatenate([x4, jnp.zeros((n_pad, 1), jnp.float32)],
                         axis=1).reshape(nx, 128)

    rWd = g_rW0[:, :, 0:12]
    rWs = g_rW0[:, :, 12:24]
    rWe = g_rW0[:, :, 24:36]
    oWn = g_oW0[:, :, 0:12]
    oWa = g_oW0[:, :, 12:24]
    wWd = w_W0[:, 0:12]
    wWs = w_W0[:, 12:24]
    wWe = w_W0[:, 24:36]

    cs, cbs = [], []
    for step in range(8):
        l = step % 4
        if step == 0:
            cs.append(_DOT(rWe[0], ee_W1))
            cbs.append(_DOT(ee_b1, rWe[0].T) + g_rb0[0])
        else:
            lp = (step - 1) % 4
            cs.append(_DOT(rWe[l], g_rW1[lp]))
            cbs.append(_DOT(g_rb1[lp], rWe[l].T) + g_rb0[l])
    cw = _DOT(wWe, g_rW1[3])
    cbw = _DOT(g_rb1[3], wWe.T) + w_b0

    enc_m0 = _grp(ee_W0.T, 4, 8, 32)
    enc_b0 = _pad_tile(ee_b0, 8, 32)
    enc_m1 = _grp(cs[0].T, 8, 16, 32)
    enc_b1 = _qvec(cbs[0], 32)
    nenc_m0 = _grp(ne_W0.T, 4, 8, 32)
    nenc_b0 = _pad_tile(ne_b0, 8, 32)
    nenc_m1 = _grp(ne_W1.T, 8, 16, 32)
    nenc_b1 = _pad_tile(ne_b1, 16, 32)
    qm = [_grp(cs[s].T) for s in range(1, 8)] + [_grp(cw.T)]
    qb = [_qvec(cbs[s]) for s in range(1, 8)] + [_qvec(cbw)]
    mds, mss = [], []
    for l in range(4):
        mds.append(_grp(rWd[l].T, 16, 16))
        mss.append(_grp(rWs[l].T, 16, 16))
    mdw = _grp(wWd.T, 16, 16)
    msw = _grp(wWs.T, 16, 16)
    msos, mns, m1s, bhs, bns = [], [], [], [], []
    for l in range(4):
        ma = jnp.zeros((16, 16), jnp.float32)
        ma = ma.at[:8, :12].set(g_rW1[l].T)
        ma = ma.at[8, :12].set(g_rb1[l])
        mo = jnp.zeros((16, 16), jnp.float32).at[:12, :8].set(oWa[l].T)
        msos.append(jnp.kron(jnp.eye(8, dtype=jnp.float32), _DOT(ma, mo)))
        mns.append(_grp(oWn[l].T, 16, 16))
        m1s.append(_grp(g_oW1[l].T, 16, 16))
        bhs.append(_pad_tile(g_ob0[l], 16, 8))
        bns.append(_pad_tile(g_ob1[l], 16, 8))
    mw = jnp.kron(jnp.eye(8, dtype=jnp.float32),
                  jnp.zeros((16, 1), jnp.float32).at[:8, 0].set(w_W1[0]))
    bw = jnp.broadcast_to(w_b1.reshape(1, 1), (1, 8))

    q = _enc_edge(ea_v, enc_m0, enc_b0, enc_m1, enc_b1,
                  ea_rows=ea_rows).reshape(e_pad, _F)
    node, pd, ps = _enc_node(x4, nenc_m0, nenc_b0, nenc_m1, nenc_b1,
                             _grp(rWd[0].T, 16, 16, 32),
                             _grp(rWs[0].T, 16, 16, 32), nx_rows=nx)
    node = node.reshape(nq, 128)
    pd = pd.reshape(n_pad, _F)
    ps = ps.reshape(n_pad, _F)
    for step in range(8):
        l = step % 4
        h, S = _sc_edge_pass(dst_rows, src_rows, q, pd, ps,
                             n_pad=n_pad, e_pad=e_pad, iters=iters)
        nl = (step + 1) % 4
        md = mds[nl] if step < 7 else mdw
        ms = mss[nl] if step < 7 else msw
        node, pd, ps = _node_update(
            node, S[:n_pad].reshape(nq, 128), S[n_pad:].reshape(nq, 128),
            msos[l], mns[l], m1s[l], bhs[l], bns[l], md, ms, nq_rows=nq)
        pd = pd.reshape(n_pad, _F)
        ps = ps.reshape(n_pad, _F)
        q = _q_stream(h.reshape(eq, 128), qm[step], qb[step],
                      eq_rows=eq).reshape(e_pad, _F)
    hw, _ = _sc_edge_pass(dst_rows, src_rows, q, pd, ps,
                          n_pad=n_pad, e_pad=e_pad, iters=iters)
    out = _head(hw.reshape(eq, 128), mw, bw, eq_rows=eq)
    return out.reshape(e_pad, 1)[:e]
